# baseline (speedup 1.0000x reference)
import functools

import jax
import jax.numpy as jnp
from jax import lax
from jax.experimental import pallas as pl
from jax.experimental.pallas import tpu as pltpu, tpu_sc as plsc

D = 1024
SCALE = float(D) ** 0.5
NC = 2
NS = 16
NW = NC * NS
LANES = 16


def _make_emb_kernel(n_rows: int):
    per_w = n_rows // NW
    chunk = 32
    nchunk = per_w // chunk

    mesh = plsc.VectorSubcoreMesh(
        core_axis_name="c", subcore_axis_name="s",
        num_cores=NC, num_subcores=NS,
    )

    @functools.partial(
        pl.kernel,
        out_type=jax.ShapeDtypeStruct((n_rows, D), jnp.float32),
        mesh=mesh,
        compiler_params=pltpu.CompilerParams(needs_layout_passes=False),
        scratch_types=[
            pltpu.VMEM((per_w,), jnp.int32),
            pltpu.VMEM((chunk, D), jnp.float32),
            pltpu.VMEM((chunk, D), jnp.float32),
            pltpu.VMEM((chunk, D), jnp.float32),
            pltpu.SemaphoreType.DMA,
            pltpu.SemaphoreType.DMA,
            pltpu.SemaphoreType.DMA,
            pltpu.SemaphoreType.DMA,
            pltpu.SemaphoreType.DMA,
            pltpu.SemaphoreType.DMA,
        ],
    )
    def emb(idx_hbm, table_hbm, out_hbm, idx_v, buf0, buf1, buf2,
            gsem0, gsem1, gsem2, osem0, osem1, osem2):
        wid = lax.axis_index("s") * NC + lax.axis_index("c")
        base = wid * per_w
        pltpu.sync_copy(idx_hbm.at[pl.ds(base, per_w)], idx_v)

        nbuf = 3
        bufs = (buf0, buf1, buf2)
        gsems = (gsem0, gsem1, gsem2)
        osems = (osem0, osem1, osem2)

        def start_gather(c):
            p = c % nbuf
            return pltpu.async_copy(
                table_hbm.at[idx_v.at[pl.ds(c * chunk, chunk)]],
                bufs[p], gsems[p])

        def start_out(c):
            p = c % nbuf
            return pltpu.async_copy(
                bufs[p], out_hbm.at[pl.ds(base + c * chunk, chunk)],
                osems[p])

        def compute(c):
            buf = bufs[c % nbuf]

            def grp_body(g, carry):
                base_r = g * LANES
                iv = idx_v[pl.ds(c * chunk + base_r, LANES)]
                sv = jnp.where(iv == 0, 0.0, SCALE).astype(jnp.float32)

                def row_body(rr, carry2):
                    bc = jnp.take_along_axis(
                        sv, jnp.full((LANES,), rr, jnp.int32), axis=0)
                    r = base_r + rr
                    for j in range(D // LANES):
                        buf[r, pl.ds(j * LANES, LANES)] = (
                            buf[r, pl.ds(j * LANES, LANES)] * bc)
                    return carry2

                lax.fori_loop(0, LANES, row_body, 0)
                return carry

            lax.fori_loop(0, chunk // LANES, grp_body, 0)

        ghandles = [None] * nbuf
        ohandles = [None] * nbuf
        ghandles[0] = start_gather(0)
        ghandles[1] = start_gather(1)
        for c in range(nchunk):
            p = c % nbuf
            ghandles[p].wait()
            if c + 2 < nchunk:
                q = (c + 2) % nbuf
                if ohandles[q] is not None:
                    ohandles[q].wait()
                ghandles[q] = start_gather(c + 2)
            compute(c)
            ohandles[p] = start_out(c)
        for h in ohandles:
            if h is not None:
                h.wait()

    return emb


@jax.jit
def kernel(inputs, shared_weights):
    b, s = inputs.shape
    n = b * s
    flat_idx = inputs.reshape(n).astype(jnp.int32)
    emb = _make_emb_kernel(n)
    out = emb(flat_idx, shared_weights)
    return out.reshape(b, s, shared_weights.shape[1])


# speedup vs baseline: 1.0664x; 1.0664x over previous
import functools

import jax
import jax.numpy as jnp
from jax import lax
from jax.experimental import pallas as pl
from jax.experimental.pallas import tpu as pltpu, tpu_sc as plsc

D = 1024
SCALE = float(D) ** 0.5
NC = 2
NS = 16
NW = NC * NS
LANES = 16


def _make_emb_kernel(batch: int, seq: int):
    n_rows = batch * seq
    per_w = n_rows // NW
    w_per_b = seq // per_w
    chunk = 32
    nchunk = per_w // chunk
    ngrp = chunk // LANES

    mesh = plsc.VectorSubcoreMesh(
        core_axis_name="c", subcore_axis_name="s",
        num_cores=NC, num_subcores=NS,
    )

    @functools.partial(
        pl.kernel,
        out_type=jax.ShapeDtypeStruct((batch, seq, D), jnp.float32),
        mesh=mesh,
        compiler_params=pltpu.CompilerParams(needs_layout_passes=False),
        scratch_types=[
            pltpu.VMEM((per_w,), jnp.int32),
            pltpu.VMEM((chunk, D), jnp.float32),
            pltpu.VMEM((chunk, D), jnp.float32),
            pltpu.VMEM((chunk, D), jnp.float32),
            pltpu.SemaphoreType.DMA,
            pltpu.SemaphoreType.DMA,
            pltpu.SemaphoreType.DMA,
            pltpu.SemaphoreType.DMA,
            pltpu.SemaphoreType.DMA,
            pltpu.SemaphoreType.DMA,
        ],
    )
    def emb(idx_hbm, table_hbm, out_hbm, idx_v, buf0, buf1, buf2,
            gsem0, gsem1, gsem2, osem0, osem1, osem2):
        wid = lax.axis_index("s") * NC + lax.axis_index("c")
        base = wid * per_w
        b_i = wid // w_per_b
        s_base = (wid % w_per_b) * per_w
        pltpu.sync_copy(idx_hbm.at[pl.ds(base, per_w)], idx_v)

        nbuf = 3
        bufs = (buf0, buf1, buf2)
        gsems = (gsem0, gsem1, gsem2)
        osems = (osem0, osem1, osem2)

        def start_gather(c):
            p = c % nbuf
            return pltpu.async_copy(
                table_hbm.at[idx_v.at[pl.ds(c * chunk, chunk)]],
                bufs[p], gsems[p])

        def start_out_grp(c, g):
            p = c % nbuf
            return pltpu.async_copy(
                bufs[p].at[pl.ds(g * LANES, LANES)],
                out_hbm.at[b_i, pl.ds(s_base + c * chunk + g * LANES, LANES)],
                osems[p])

        def compute_grp(c, g):
            buf = bufs[c % nbuf]
            base_r = g * LANES
            iv = idx_v[pl.ds(c * chunk + base_r, LANES)]
            sv = jnp.where(iv == 0, 0.0, SCALE).astype(jnp.float32)

            def row_body(rr, carry):
                bc = jnp.take_along_axis(
                    sv, jnp.full((LANES,), rr, jnp.int32), axis=0)
                r = base_r + rr
                for j in range(D // LANES):
                    buf[r, pl.ds(j * LANES, LANES)] = (
                        buf[r, pl.ds(j * LANES, LANES)] * bc)
                return carry

            lax.fori_loop(0, LANES, row_body, 0)

        ghandles = [None] * nbuf
        ohandles = [[None] * ngrp for _ in range(nbuf)]
        ghandles[0] = start_gather(0)
        ghandles[1] = start_gather(1)
        for c in range(nchunk):
            p = c % nbuf
            ghandles[p].wait()
            if c + 2 < nchunk:
                q = (c + 2) % nbuf
                for h in ohandles[q]:
                    if h is not None:
                        h.wait()
                ohandles[q] = [None] * ngrp
                ghandles[q] = start_gather(c + 2)
            for g in range(ngrp):
                compute_grp(c, g)
                ohandles[p][g] = start_out_grp(c, g)
        for hs in ohandles:
            for h in hs:
                if h is not None:
                    h.wait()

    return emb


@jax.jit
def kernel(inputs, shared_weights):
    b, s = inputs.shape
    flat_idx = inputs.reshape(b * s).astype(jnp.int32)
    emb = _make_emb_kernel(b, s)
    return emb(flat_idx, shared_weights)
